# baseline (speedup 1.0000x reference)
import functools

import jax
import jax.numpy as jnp
from jax import lax
from jax.experimental import pallas as pl
from jax.experimental.pallas import tpu as pltpu
from jax.experimental.pallas import tpu_sc as plsc

_BATCH = 16384
_DIM = 64
_NW = 32
_BPW = _BATCH // _NW
_IDX_CHUNK = 128
_NCHUNK = _BPW // _IDX_CHUNK
_LANES = 16


def _cbow_body(xf_hbm, a_hbm, b_hbm, out_hbm, xbuf, idx_v, rows0, rows1,
               rows2, out_v, sem):
    wid = lax.axis_index("s") * 2 + lax.axis_index("c")
    base = wid * _BPW

    pltpu.sync_copy(xf_hbm.at[pl.ds(base * 3, _BPW * 3)], xbuf)

    lane3 = lax.iota(jnp.int32, _LANES) * 3
    for t in range(3):
        for g in range(_BPW // _LANES):
            vals = plsc.load_gather(xbuf, [lane3 + (48 * g + t)])
            idx_v[t * _NCHUNK + g // 8, pl.ds((g % 8) * _LANES, _LANES)] = vals

    copies = []
    for t, rows in enumerate((rows0, rows1, rows2)):
        for j in range(_NCHUNK):
            copies.append(
                pltpu.async_copy(
                    (a_hbm if t < 2 else b_hbm).at[idx_v.at[t * _NCHUNK + j]],
                    rows.at[pl.ds(j * _IDX_CHUNK, _IDX_CHUNK)],
                    sem,
                )
            )
    for c in copies:
        c.wait()

    lane_iota = lax.iota(jnp.int32, _LANES)

    def group_body(g, _):
        r = g * _LANES
        row_ids = r + lane_iota
        acc = jnp.zeros((_LANES,), jnp.float32)
        for d in range(_DIM):
            col = jnp.full((_LANES,), d, jnp.int32)
            a0 = plsc.load_gather(rows0, [row_ids, col])
            a1 = plsc.load_gather(rows1, [row_ids, col])
            bv = plsc.load_gather(rows2, [row_ids, col])
            acc = acc + (a0 + a1) * bv
        out_v[pl.ds(r, _LANES)] = acc * 0.5
        return 0

    lax.fori_loop(0, _BPW // _LANES, group_body, 0)

    pltpu.sync_copy(out_v, out_hbm.at[pl.ds(base, _BPW)])


@jax.jit
def _cbow(xf, A, B):
    mesh = plsc.VectorSubcoreMesh(core_axis_name="c", subcore_axis_name="s")
    f = pl.kernel(
        _cbow_body,
        out_type=jax.ShapeDtypeStruct((_BATCH,), jnp.float32),
        mesh=mesh,
        scratch_types=[
            pltpu.VMEM((_BPW * 3,), jnp.int32),
            pltpu.VMEM((3 * _NCHUNK, _IDX_CHUNK), jnp.int32),
            pltpu.VMEM((_BPW, _DIM), jnp.float32),
            pltpu.VMEM((_BPW, _DIM), jnp.float32),
            pltpu.VMEM((_BPW, _DIM), jnp.float32),
            pltpu.VMEM((_BPW,), jnp.float32),
            pltpu.SemaphoreType.DMA,
        ],
        compiler_params=pltpu.CompilerParams(
            needs_layout_passes=False, use_tc_tiling_on_sc=False
        ),
    )
    return f(xf, A, B)


def kernel(x, A, B):
    xf = jnp.abs(x.astype(jnp.int32).reshape(-1))
    return _cbow(xf, A, B)


# speedup vs baseline: 1.3465x; 1.3465x over previous
import functools

import jax
import jax.numpy as jnp
import jax.experimental.layout as jlayout
from jax import lax
from jax.experimental import pallas as pl
from jax.experimental.pallas import tpu as pltpu
from jax.experimental.pallas import tpu_sc as plsc

_BATCH = 16384
_DIM = 64
_NW = 32
_BPW = _BATCH // _NW
_IDX_CHUNK = 128
_NCHUNK = _BPW // _IDX_CHUNK
_LANES = 16


def _cbow_body(xf_hbm, a_hbm, b_hbm, out_hbm, xbuf, idx_v, rows0, rows1,
               rows2, stage, out_v, sem):
    wid = lax.axis_index("s") * 2 + lax.axis_index("c")
    base = wid * _BPW

    pltpu.sync_copy(xf_hbm.at[pl.ds(base * 3, _BPW * 3)], xbuf)

    lane3 = lax.iota(jnp.int32, _LANES) * 3
    for t in range(3):
        for g in range(_BPW // _LANES):
            vals = plsc.load_gather(xbuf, [lane3 + (48 * g + t)])
            idx_v[t * _NCHUNK + g // 8, pl.ds((g % 8) * _LANES, _LANES)] = vals

    copies = []
    for j in range(_NCHUNK):
        for t, rows in enumerate((rows0, rows1, rows2)):
            copies.append(
                pltpu.async_copy(
                    (a_hbm if t < 2 else b_hbm).at[idx_v.at[t * _NCHUNK + j]],
                    rows.at[pl.ds(j * _IDX_CHUNK, _IDX_CHUNK)],
                    sem,
                )
            )

    lane_iota = lax.iota(jnp.int32, _LANES)
    lane17 = lane_iota * 17

    def group_body(g, _):
        r = g * _LANES
        for l in range(_LANES):
            vacc = jnp.zeros((_LANES,), jnp.float32)
            for c in range(_DIM // _LANES):
                a0 = rows0[r + l, pl.ds(c * _LANES, _LANES)]
                a1 = rows1[r + l, pl.ds(c * _LANES, _LANES)]
                bv = rows2[r + l, pl.ds(c * _LANES, _LANES)]
                vacc = vacc + (a0 + a1) * bv
            stage[pl.ds(l * 17, _LANES)] = vacc
        acc = jnp.zeros((_LANES,), jnp.float32)
        for c in range(_LANES):
            acc = acc + plsc.load_gather(stage, [lane17 + c])
        out_v[pl.ds(r, _LANES)] = acc * 0.5
        return 0

    groups_per_chunk = _IDX_CHUNK // _LANES
    for j in range(_NCHUNK):
        for t in range(3):
            copies[j * 3 + t].wait()
        lax.fori_loop(
            j * groups_per_chunk, (j + 1) * groups_per_chunk, group_body, 0
        )

    pltpu.sync_copy(out_v, out_hbm.at[pl.ds(base, _BPW)])


_LINEAR_2D = jlayout.Layout(major_to_minor=(0, 1), tiling=())


@jax.jit
def _cbow(xf, A, B):
    mesh = plsc.VectorSubcoreMesh(core_axis_name="c", subcore_axis_name="s")
    f = pl.kernel(
        _cbow_body,
        out_type=jax.ShapeDtypeStruct((_BATCH,), jnp.float32),
        mesh=mesh,
        scratch_types=[
            pltpu.VMEM((_BPW * 3,), jnp.int32),
            pltpu.VMEM((3 * _NCHUNK, _IDX_CHUNK), jnp.int32),
            pltpu.VMEM((_BPW, _DIM), jnp.float32),
            pltpu.VMEM((_BPW, _DIM), jnp.float32),
            pltpu.VMEM((_BPW, _DIM), jnp.float32),
            pltpu.VMEM((_LANES * 17,), jnp.float32),
            pltpu.VMEM((_BPW,), jnp.float32),
            pltpu.SemaphoreType.DMA,
        ],
        compiler_params=pltpu.CompilerParams(
            needs_layout_passes=False, use_tc_tiling_on_sc=False
        ),
    )
    return f(xf, A, B)


def kernel(x, A, B):
    xf = jnp.abs(x.astype(jnp.int32).reshape(-1))
    return _cbow(xf, A, B)
